# baseline (speedup 1.0000x reference)
import functools

import jax
import jax.numpy as jnp
from jax import lax
from jax.experimental import pallas as pl
from jax.experimental.pallas import tpu as pltpu
from jax.experimental.pallas import tpu_sc as plsc

_H = 128


def _atom_body(x_ref, w_ref, b_ref, out_ref, asum_ref):
    v = jnp.dot(x_ref[...], w_ref[...], preferred_element_type=jnp.float32)
    v = v + b_ref[...]
    h = v.shape[1] // 2
    lo = jax.lax.bitcast_convert_type(
        v[:, :h].astype(jnp.bfloat16).astype(jnp.float32), jnp.uint32)
    hi = jax.lax.bitcast_convert_type(
        v[:, h:].astype(jnp.bfloat16).astype(jnp.float32), jnp.uint32)
    packed = jax.lax.shift_right_logical(lo, jnp.uint32(16)) | hi
    out_ref[...] = jax.lax.bitcast_convert_type(packed, jnp.float32)

    @pl.when(pl.program_id(0) == 0)
    def _():
        asum_ref[...] = jnp.zeros_like(asum_ref)

    asum_ref[...] += jnp.sum(v, axis=0, keepdims=True)


def _atom_encoder(atom_features, WaT, b_atom):
    n, fdim = atom_features.shape
    tile = 2000
    grid = (n // tile,)
    return pl.pallas_call(
        _atom_body,
        grid=grid,
        in_specs=[
            pl.BlockSpec((tile, fdim), lambda i: (i, 0)),
            pl.BlockSpec((fdim, _H), lambda i: (0, 0)),
            pl.BlockSpec((1, _H), lambda i: (0, 0)),
        ],
        out_specs=[
            pl.BlockSpec((tile, _H // 2), lambda i: (i, 0)),
            pl.BlockSpec((1, _H), lambda i: (0, 0)),
        ],
        out_shape=[
            jax.ShapeDtypeStruct((n, _H // 2), jnp.float32),
            jax.ShapeDtypeStruct((1, _H), jnp.float32),
        ],
        compiler_params=pltpu.CompilerParams(dimension_semantics=("arbitrary",)),
    )(atom_features, WaT, b_atom.reshape(1, _H))


def _gather_rows(table, idx):
    n_rows = idx.shape[0]
    width = table.shape[1]
    dtype = table.dtype
    ncores, nsub = 2, 16
    nw = ncores * nsub
    ch = 128
    n_chunks = n_rows // ch
    base_cnt = n_chunks // nw
    rem = n_chunks - base_cnt * nw
    maxc = base_cnt + (1 if rem else 0)
    half_iters = (maxc + 1) // 2
    mesh = plsc.VectorSubcoreMesh(core_axis_name="c", subcore_axis_name="s")

    @functools.partial(
        pl.kernel,
        mesh=mesh,
        compiler_params=pltpu.CompilerParams(use_tc_tiling_on_sc=False),
        out_type=jax.ShapeDtypeStruct((n_rows, width), dtype),
        scratch_types=[
            pltpu.VMEM((maxc * ch,), jnp.int32),
            pltpu.VMEM((2, ch, width), dtype),
            pltpu.SemaphoreType.DMA,
            pltpu.SemaphoreType.DMA,
            pltpu.SemaphoreType.DMA,
        ],
    )
    def gather_k(table_hbm, idx_hbm, out_hbm, idx_v, rows_v, sem_g, sem_w0,
                 sem_w1):
        wid = lax.axis_index("c") * nsub + lax.axis_index("s")
        cnt = base_cnt + jnp.where(wid < rem, 1, 0)
        c0 = wid * base_cnt + jnp.minimum(wid, rem)

        pltpu.sync_copy(idx_hbm.at[pl.ds(c0 * ch, base_cnt * ch)],
                        idx_v.at[pl.ds(0, base_cnt * ch)])

        @pl.when(wid < rem)
        def _():
            pltpu.sync_copy(idx_hbm.at[pl.ds((c0 + base_cnt) * ch, ch)],
                            idx_v.at[pl.ds(base_cnt * ch, ch)])

        def half(i, slot, sem_w):
            @pl.when(i < cnt)
            def _():
                rows = rows_v.at[slot]

                @pl.when(i >= 2)
                def _():
                    pltpu.make_async_copy(rows, out_hbm.at[pl.ds(0, ch)],
                                          sem_w).wait()

                pltpu.async_copy(
                    table_hbm.at[idx_v.at[pl.ds(i * ch, ch)]], rows,
                    sem_g).wait()
                pltpu.async_copy(rows, out_hbm.at[pl.ds((c0 + i) * ch, ch)],
                                 sem_w)

        def body(j, carry):
            half(2 * j, 0, sem_w0)
            half(2 * j + 1, 1, sem_w1)
            return carry

        lax.fori_loop(0, half_iters, body, 0)
        pltpu.make_async_copy(rows_v.at[0], out_hbm.at[pl.ds(0, ch)],
                              sem_w0).wait()
        pltpu.make_async_copy(rows_v.at[1], out_hbm.at[pl.ds(0, ch)],
                              sem_w1).wait()

    return gather_k(table, idx)


def _mpnn_body(depth, g_ref, bond_ref, wbT_ref, bb_ref, wc_ref, wit_ref,
               bgiA_ref, bhhn_ref, msum_ref):
    wit = wit_ref[...]
    msg = jnp.dot(bond_ref[...], wbT_ref[...], preferred_element_type=jnp.float32)
    msg = msg + bb_ref[...]
    gp = jax.lax.bitcast_convert_type(g_ref[...], jnp.uint32)
    g_lo = jax.lax.bitcast_convert_type(
        jax.lax.shift_left(gp, jnp.uint32(16)), jnp.float32)
    g_hi = jax.lax.bitcast_convert_type(gp & jnp.uint32(0xFFFF0000),
                                        jnp.float32)
    g = jnp.concatenate([g_lo, g_hi], axis=1)
    giA = jnp.dot(g, wc_ref[...],
                  preferred_element_type=jnp.float32) + bgiA_ref[...]
    bhhn = bhhn_ref[...]
    for _ in range(depth):
        t = jnp.dot(msg, wit, preferred_element_type=jnp.float32)
        s_rz = giA[:, : 2 * _H] + t[:, : 2 * _H]
        r = 0.5 + 0.5 * jnp.tanh(0.5 * s_rz[:, :_H])
        z = 0.5 + 0.5 * jnp.tanh(0.5 * s_rz[:, _H:])
        nv = jnp.tanh(giA[:, 2 * _H:] + t[:, 2 * _H: 3 * _H]
                      + r * (t[:, 3 * _H:] + bhhn))
        msg = nv + z * (msg - nv)

    @pl.when(pl.program_id(0) == 0)
    def _():
        msum_ref[...] = jnp.zeros_like(msum_ref)

    msum_ref[...] += jnp.sum(msg, axis=0, keepdims=True)


def _mpnn(gathered, bond_features, WbT, b_bond, Wc, Wit, bgiA, bhhn, depth):
    e, bdim = bond_features.shape
    tile = 2000
    grid = (e // tile,)
    return pl.pallas_call(
        functools.partial(_mpnn_body, depth),
        grid=grid,
        in_specs=[
            pl.BlockSpec((tile, _H // 2), lambda i: (i, 0)),
            pl.BlockSpec((tile, bdim), lambda i: (i, 0)),
            pl.BlockSpec((bdim, _H), lambda i: (0, 0)),
            pl.BlockSpec((1, _H), lambda i: (0, 0)),
            pl.BlockSpec((_H, 3 * _H), lambda i: (0, 0)),
            pl.BlockSpec((_H, 4 * _H), lambda i: (0, 0)),
            pl.BlockSpec((1, 3 * _H), lambda i: (0, 0)),
            pl.BlockSpec((1, _H), lambda i: (0, 0)),
        ],
        out_specs=pl.BlockSpec((1, _H), lambda i: (0, 0)),
        out_shape=jax.ShapeDtypeStruct((1, _H), jnp.float32),
        compiler_params=pltpu.CompilerParams(dimension_semantics=("arbitrary",)),
    )(gathered, bond_features, WbT, b_bond.reshape(1, _H),
      Wc, Wit, bgiA.reshape(1, 3 * _H), bhhn.reshape(1, _H))


def _readout_body(inv_n, asum_ref, msum_ref, wr1_ref, br1_ref, wr2_ref, br2_ref,
                  ws1_ref, bs1_ref, ws2_ref, bs2_ref, out_ref):
    msum = jnp.sum(msum_ref[...], axis=0, keepdims=True)
    mol = (asum_ref[...] + msum) * inv_n
    mol8 = jnp.broadcast_to(mol, (8, _H))
    h = jnp.dot(mol8, wr1_ref[...], preferred_element_type=jnp.float32)
    h = jnp.maximum(h + br1_ref[...], 0.0)
    m2 = jnp.dot(h, wr2_ref[...], preferred_element_type=jnp.float32) + br2_ref[...]
    h2 = jnp.dot(m2, ws1_ref[...], preferred_element_type=jnp.float32)
    h2 = jnp.maximum(h2 + bs1_ref[...], 0.0)
    sp = jnp.dot(h2, ws2_ref[...], preferred_element_type=jnp.float32) + bs2_ref[...]
    out_ref[...] = sp[0:1, :]


def _readout(asum, msum, n, W_r1, b_r1, W_r2, b_r2, W_s1, b_s1, W_s2, b_s2):
    spec_dim = W_s2.shape[0]
    h2 = W_s1.shape[0]
    return pl.pallas_call(
        functools.partial(_readout_body, 1.0 / n),
        in_specs=[pl.BlockSpec(a.shape, lambda: tuple(0 for _ in a.shape))
                  for a in (asum, msum, W_r1.T, b_r1.reshape(1, -1), W_r2.T,
                            b_r2.reshape(1, -1), W_s1.T, b_s1.reshape(1, -1),
                            W_s2.T, b_s2.reshape(1, -1))],
        out_specs=pl.BlockSpec((1, spec_dim), lambda: (0, 0)),
        out_shape=jax.ShapeDtypeStruct((1, spec_dim), jnp.float32),
    )(asum, msum, W_r1.T, b_r1.reshape(1, -1), W_r2.T, b_r2.reshape(1, -1),
      W_s1.T, b_s1.reshape(1, -1), W_s2.T, b_s2.reshape(1, -1))


def kernel(atom_features, bond_features, edge_src, edge_dst,
           W_atom, b_atom, W_bond, b_bond, W_msg, b_msg,
           gru_wih, gru_whh, gru_bih, gru_bhh,
           W_r1, b_r1, W_r2, b_r2, W_s1, b_s1, W_s2, b_s2):
    n = atom_features.shape[0]
    depth = 3

    Wc = W_msg.T @ gru_wih.T
    bc = b_msg @ gru_wih.T + gru_bih
    WhhT = gru_whh.T
    Wit = jnp.concatenate(
        [Wc[:, : 2 * _H] + WhhT[:, : 2 * _H], Wc[:, 2 * _H:],
         WhhT[:, 2 * _H:]], axis=1)
    bgiA = jnp.concatenate(
        [bc[: 2 * _H] + gru_bhh[: 2 * _H], bc[2 * _H:]])
    bhhn = gru_bhh[2 * _H:]

    av_packed, asum = _atom_encoder(atom_features, W_atom.T, b_atom)
    gathered = _gather_rows(av_packed, edge_src)
    msum = _mpnn(gathered, bond_features, W_bond.T, b_bond, Wc, Wit, bgiA,
                 bhhn, depth)
    return _readout(asum, msum, n, W_r1, b_r1, W_r2, b_r2,
                    W_s1, b_s1, W_s2, b_s2)


# speedup vs baseline: 1.1128x; 1.1128x over previous
import functools

import jax
import jax.numpy as jnp
from jax import lax
from jax.experimental import pallas as pl
from jax.experimental.pallas import tpu as pltpu
from jax.experimental.pallas import tpu_sc as plsc

_H = 128


def _atom_body(x_ref, w_ref, b_ref, out_ref, asum_ref):
    v = jnp.dot(x_ref[...], w_ref[...], preferred_element_type=jnp.float32)
    v = v + b_ref[...]
    out_ref[...] = v

    @pl.when(pl.program_id(0) == 0)
    def _():
        asum_ref[...] = jnp.zeros_like(asum_ref)

    asum_ref[...] += jnp.sum(v, axis=0, keepdims=True)


def _atom_encoder(atom_features, WaT, b_atom):
    n, fdim = atom_features.shape
    tile = 2000
    grid = (n // tile,)
    return pl.pallas_call(
        _atom_body,
        grid=grid,
        in_specs=[
            pl.BlockSpec((tile, fdim), lambda i: (i, 0)),
            pl.BlockSpec((fdim, _H), lambda i: (0, 0)),
            pl.BlockSpec((1, _H), lambda i: (0, 0)),
        ],
        out_specs=[
            pl.BlockSpec((tile, _H), lambda i: (i, 0)),
            pl.BlockSpec((1, _H), lambda i: (0, 0)),
        ],
        out_shape=[
            jax.ShapeDtypeStruct((n, _H), jnp.float32),
            jax.ShapeDtypeStruct((1, _H), jnp.float32),
        ],
        compiler_params=pltpu.CompilerParams(dimension_semantics=("arbitrary",)),
    )(atom_features, WaT, b_atom.reshape(1, _H))


def _gather_rows(table, idx):
    n_rows = idx.shape[0]
    width = table.shape[1]
    dtype = table.dtype
    ncores, nsub = 2, 16
    nw = ncores * nsub
    ch = 128
    n_chunks = n_rows // ch
    base_cnt = n_chunks // nw
    rem = n_chunks - base_cnt * nw
    maxc = base_cnt + (1 if rem else 0)
    half_iters = (maxc + 1) // 2
    mesh = plsc.VectorSubcoreMesh(core_axis_name="c", subcore_axis_name="s")

    @functools.partial(
        pl.kernel,
        mesh=mesh,
        out_type=jax.ShapeDtypeStruct((n_rows, width), dtype),
        scratch_types=[
            pltpu.VMEM((maxc * ch,), jnp.int32),
            pltpu.VMEM((2, ch, width), dtype),
            pltpu.SemaphoreType.DMA,
            pltpu.SemaphoreType.DMA,
            pltpu.SemaphoreType.DMA,
        ],
    )
    def gather_k(table_hbm, idx_hbm, out_hbm, idx_v, rows_v, sem_g, sem_w0,
                 sem_w1):
        wid = lax.axis_index("c") * nsub + lax.axis_index("s")
        cnt = base_cnt + jnp.where(wid < rem, 1, 0)
        c0 = wid * base_cnt + jnp.minimum(wid, rem)

        pltpu.sync_copy(idx_hbm.at[pl.ds(c0 * ch, base_cnt * ch)],
                        idx_v.at[pl.ds(0, base_cnt * ch)])

        @pl.when(wid < rem)
        def _():
            pltpu.sync_copy(idx_hbm.at[pl.ds((c0 + base_cnt) * ch, ch)],
                            idx_v.at[pl.ds(base_cnt * ch, ch)])

        def half(i, slot, sem_w):
            @pl.when(i < cnt)
            def _():
                rows = rows_v.at[slot]

                @pl.when(i >= 2)
                def _():
                    pltpu.make_async_copy(rows, out_hbm.at[pl.ds(0, ch)],
                                          sem_w).wait()

                pltpu.async_copy(
                    table_hbm.at[idx_v.at[pl.ds(i * ch, ch)]], rows,
                    sem_g).wait()
                pltpu.async_copy(rows, out_hbm.at[pl.ds((c0 + i) * ch, ch)],
                                 sem_w)

        def body(j, carry):
            half(2 * j, 0, sem_w0)
            half(2 * j + 1, 1, sem_w1)
            return carry

        lax.fori_loop(0, half_iters, body, 0)
        pltpu.make_async_copy(rows_v.at[0], out_hbm.at[pl.ds(0, ch)],
                              sem_w0).wait()
        pltpu.make_async_copy(rows_v.at[1], out_hbm.at[pl.ds(0, ch)],
                              sem_w1).wait()

    return gather_k(table, idx)


def _mpnn_body(depth, g_ref, bond_ref, wbT_ref, bb_ref, wc_ref, wit_ref,
               bgiA_ref, bhhn_ref, msum_ref):
    wit = wit_ref[...]
    msg = jnp.dot(bond_ref[...], wbT_ref[...], preferred_element_type=jnp.float32)
    msg = msg + bb_ref[...]
    giA = jnp.dot(g_ref[...], wc_ref[...],
                  preferred_element_type=jnp.float32) + bgiA_ref[...]
    bhhn = bhhn_ref[...]
    for _ in range(depth):
        t = jnp.dot(msg, wit, preferred_element_type=jnp.float32)
        s_rz = giA[:, : 2 * _H] + t[:, : 2 * _H]
        r = 0.5 + 0.5 * jnp.tanh(0.5 * s_rz[:, :_H])
        z = 0.5 + 0.5 * jnp.tanh(0.5 * s_rz[:, _H:])
        nv = jnp.tanh(giA[:, 2 * _H:] + t[:, 2 * _H: 3 * _H]
                      + r * (t[:, 3 * _H:] + bhhn))
        msg = nv + z * (msg - nv)

    @pl.when(pl.program_id(0) == 0)
    def _():
        msum_ref[...] = jnp.zeros_like(msum_ref)

    msum_ref[...] += jnp.sum(msg, axis=0, keepdims=True)


def _mpnn(gathered, bond_features, WbT, b_bond, Wc, Wit, bgiA, bhhn, depth):
    e, bdim = bond_features.shape
    tile = 2000
    grid = (e // tile,)
    return pl.pallas_call(
        functools.partial(_mpnn_body, depth),
        grid=grid,
        in_specs=[
            pl.BlockSpec((tile, _H), lambda i: (i, 0)),
            pl.BlockSpec((tile, bdim), lambda i: (i, 0)),
            pl.BlockSpec((bdim, _H), lambda i: (0, 0)),
            pl.BlockSpec((1, _H), lambda i: (0, 0)),
            pl.BlockSpec((_H, 3 * _H), lambda i: (0, 0)),
            pl.BlockSpec((_H, 4 * _H), lambda i: (0, 0)),
            pl.BlockSpec((1, 3 * _H), lambda i: (0, 0)),
            pl.BlockSpec((1, _H), lambda i: (0, 0)),
        ],
        out_specs=pl.BlockSpec((1, _H), lambda i: (0, 0)),
        out_shape=jax.ShapeDtypeStruct((1, _H), jnp.float32),
        compiler_params=pltpu.CompilerParams(dimension_semantics=("arbitrary",)),
    )(gathered, bond_features, WbT, b_bond.reshape(1, _H),
      Wc, Wit, bgiA.reshape(1, 3 * _H), bhhn.reshape(1, _H))


def _readout_body(inv_n, asum_ref, msum_ref, wr1_ref, br1_ref, wr2_ref, br2_ref,
                  ws1_ref, bs1_ref, ws2_ref, bs2_ref, out_ref):
    msum = jnp.sum(msum_ref[...], axis=0, keepdims=True)
    mol = (asum_ref[...] + msum) * inv_n
    mol8 = jnp.broadcast_to(mol, (8, _H))
    h = jnp.dot(mol8, wr1_ref[...], preferred_element_type=jnp.float32)
    h = jnp.maximum(h + br1_ref[...], 0.0)
    m2 = jnp.dot(h, wr2_ref[...], preferred_element_type=jnp.float32) + br2_ref[...]
    h2 = jnp.dot(m2, ws1_ref[...], preferred_element_type=jnp.float32)
    h2 = jnp.maximum(h2 + bs1_ref[...], 0.0)
    sp = jnp.dot(h2, ws2_ref[...], preferred_element_type=jnp.float32) + bs2_ref[...]
    out_ref[...] = sp[0:1, :]


def _readout(asum, msum, n, W_r1, b_r1, W_r2, b_r2, W_s1, b_s1, W_s2, b_s2):
    spec_dim = W_s2.shape[0]
    h2 = W_s1.shape[0]
    return pl.pallas_call(
        functools.partial(_readout_body, 1.0 / n),
        in_specs=[pl.BlockSpec(a.shape, lambda: tuple(0 for _ in a.shape))
                  for a in (asum, msum, W_r1.T, b_r1.reshape(1, -1), W_r2.T,
                            b_r2.reshape(1, -1), W_s1.T, b_s1.reshape(1, -1),
                            W_s2.T, b_s2.reshape(1, -1))],
        out_specs=pl.BlockSpec((1, spec_dim), lambda: (0, 0)),
        out_shape=jax.ShapeDtypeStruct((1, spec_dim), jnp.float32),
    )(asum, msum, W_r1.T, b_r1.reshape(1, -1), W_r2.T, b_r2.reshape(1, -1),
      W_s1.T, b_s1.reshape(1, -1), W_s2.T, b_s2.reshape(1, -1))


def kernel(atom_features, bond_features, edge_src, edge_dst,
           W_atom, b_atom, W_bond, b_bond, W_msg, b_msg,
           gru_wih, gru_whh, gru_bih, gru_bhh,
           W_r1, b_r1, W_r2, b_r2, W_s1, b_s1, W_s2, b_s2):
    n = atom_features.shape[0]
    depth = 3

    Wc = W_msg.T @ gru_wih.T
    bc = b_msg @ gru_wih.T + gru_bih
    WhhT = gru_whh.T
    Wit = jnp.concatenate(
        [Wc[:, : 2 * _H] + WhhT[:, : 2 * _H], Wc[:, 2 * _H:],
         WhhT[:, 2 * _H:]], axis=1)
    bgiA = jnp.concatenate(
        [bc[: 2 * _H] + gru_bhh[: 2 * _H], bc[2 * _H:]])
    bhhn = gru_bhh[2 * _H:]

    av_packed, asum = _atom_encoder(atom_features, W_atom.T, b_atom)
    gathered = _gather_rows(av_packed, edge_src)
    msum = _mpnn(gathered, bond_features, W_bond.T, b_bond, Wc, Wit, bgiA,
                 bhhn, depth)
    return _readout(asum, msum, n, W_r1, b_r1, W_r2, b_r2,
                    W_s1, b_s1, W_s2, b_s2)


# speedup vs baseline: 1.1523x; 1.0355x over previous
import functools

import jax
import jax.numpy as jnp
from jax import lax
from jax.experimental import pallas as pl
from jax.experimental.pallas import tpu as pltpu
from jax.experimental.pallas import tpu_sc as plsc

_H = 128


def _atom_body(x_ref, w_ref, b_ref, out_ref, asum_ref):
    v = jnp.dot(x_ref[...], w_ref[...], preferred_element_type=jnp.float32)
    v = v + b_ref[...]
    out_ref[...] = v

    @pl.when(pl.program_id(0) == 0)
    def _():
        asum_ref[...] = jnp.zeros_like(asum_ref)

    asum_ref[...] += jnp.sum(v.reshape(-1, 8, _H), axis=0)


def _atom_encoder(atom_features, WaT, b_atom):
    n, fdim = atom_features.shape
    tile = 2000
    grid = (n // tile,)
    return pl.pallas_call(
        _atom_body,
        grid=grid,
        in_specs=[
            pl.BlockSpec((tile, fdim), lambda i: (i, 0)),
            pl.BlockSpec((fdim, _H), lambda i: (0, 0)),
            pl.BlockSpec((1, _H), lambda i: (0, 0)),
        ],
        out_specs=[
            pl.BlockSpec((tile, _H), lambda i: (i, 0)),
            pl.BlockSpec((8, _H), lambda i: (0, 0)),
        ],
        out_shape=[
            jax.ShapeDtypeStruct((n, _H), jnp.float32),
            jax.ShapeDtypeStruct((8, _H), jnp.float32),
        ],
        compiler_params=pltpu.CompilerParams(dimension_semantics=("arbitrary",)),
    )(atom_features, WaT, b_atom.reshape(1, _H))


def _gather_rows(table, idx):
    n_rows = idx.shape[0]
    width = table.shape[1]
    dtype = table.dtype
    ncores, nsub = 2, 16
    nw = ncores * nsub
    ch = 128
    n_chunks = n_rows // ch
    base_cnt = n_chunks // nw
    rem = n_chunks - base_cnt * nw
    maxc = base_cnt + (1 if rem else 0)
    half_iters = (maxc + 1) // 2
    mesh = plsc.VectorSubcoreMesh(core_axis_name="c", subcore_axis_name="s")

    @functools.partial(
        pl.kernel,
        mesh=mesh,
        out_type=jax.ShapeDtypeStruct((n_rows, width), dtype),
        scratch_types=[
            pltpu.VMEM((maxc * ch,), jnp.int32),
            pltpu.VMEM((2, ch, width), dtype),
            pltpu.SemaphoreType.DMA,
            pltpu.SemaphoreType.DMA,
            pltpu.SemaphoreType.DMA,
        ],
    )
    def gather_k(table_hbm, idx_hbm, out_hbm, idx_v, rows_v, sem_g, sem_w0,
                 sem_w1):
        wid = lax.axis_index("c") * nsub + lax.axis_index("s")
        cnt = base_cnt + jnp.where(wid < rem, 1, 0)
        c0 = wid * base_cnt + jnp.minimum(wid, rem)

        pltpu.sync_copy(idx_hbm.at[pl.ds(c0 * ch, base_cnt * ch)],
                        idx_v.at[pl.ds(0, base_cnt * ch)])

        @pl.when(wid < rem)
        def _():
            pltpu.sync_copy(idx_hbm.at[pl.ds((c0 + base_cnt) * ch, ch)],
                            idx_v.at[pl.ds(base_cnt * ch, ch)])

        def half(i, slot, sem_w):
            @pl.when(i < cnt)
            def _():
                rows = rows_v.at[slot]

                @pl.when(i >= 2)
                def _():
                    pltpu.make_async_copy(rows, out_hbm.at[pl.ds(0, ch)],
                                          sem_w).wait()

                pltpu.async_copy(
                    table_hbm.at[idx_v.at[pl.ds(i * ch, ch)]], rows,
                    sem_g).wait()
                pltpu.async_copy(rows, out_hbm.at[pl.ds((c0 + i) * ch, ch)],
                                 sem_w)

        def body(j, carry):
            half(2 * j, 0, sem_w0)
            half(2 * j + 1, 1, sem_w1)
            return carry

        lax.fori_loop(0, half_iters, body, 0)
        pltpu.make_async_copy(rows_v.at[0], out_hbm.at[pl.ds(0, ch)],
                              sem_w0).wait()
        pltpu.make_async_copy(rows_v.at[1], out_hbm.at[pl.ds(0, ch)],
                              sem_w1).wait()

    return gather_k(table, idx)


def _mpnn_body(depth, g_ref, bond_ref, wbT_ref, bb_ref, wc_ref, wit_ref,
               bgiA_ref, bhhn_ref, msum_ref):
    wit = wit_ref[...]
    msg = jnp.dot(bond_ref[...], wbT_ref[...], preferred_element_type=jnp.float32)
    msg = msg + bb_ref[...]
    giA = jnp.dot(g_ref[...], wc_ref[...],
                  preferred_element_type=jnp.float32) + bgiA_ref[...]
    bhhn = bhhn_ref[...]
    for _ in range(depth):
        t = jnp.dot(msg, wit, preferred_element_type=jnp.float32)
        s_rz = giA[:, : 2 * _H] + t[:, : 2 * _H]
        r = 0.5 + 0.5 * jnp.tanh(0.5 * s_rz[:, :_H])
        z = 0.5 + 0.5 * jnp.tanh(0.5 * s_rz[:, _H:])
        nv = jnp.tanh(giA[:, 2 * _H:] + t[:, 2 * _H: 3 * _H]
                      + r * (t[:, 3 * _H:] + bhhn))
        msg = nv + z * (msg - nv)

    @pl.when(pl.program_id(0) == 0)
    def _():
        msum_ref[...] = jnp.zeros_like(msum_ref)

    msum_ref[...] += jnp.sum(msg.reshape(-1, 8, _H), axis=0)


def _mpnn(gathered, bond_features, WbT, b_bond, Wc, Wit, bgiA, bhhn, depth):
    e, bdim = bond_features.shape
    tile = 8000
    grid = (e // tile,)
    return pl.pallas_call(
        functools.partial(_mpnn_body, depth),
        grid=grid,
        in_specs=[
            pl.BlockSpec((tile, _H), lambda i: (i, 0)),
            pl.BlockSpec((tile, bdim), lambda i: (i, 0)),
            pl.BlockSpec((bdim, _H), lambda i: (0, 0)),
            pl.BlockSpec((1, _H), lambda i: (0, 0)),
            pl.BlockSpec((_H, 3 * _H), lambda i: (0, 0)),
            pl.BlockSpec((_H, 4 * _H), lambda i: (0, 0)),
            pl.BlockSpec((1, 3 * _H), lambda i: (0, 0)),
            pl.BlockSpec((1, _H), lambda i: (0, 0)),
        ],
        out_specs=pl.BlockSpec((8, _H), lambda i: (0, 0)),
        out_shape=jax.ShapeDtypeStruct((8, _H), jnp.float32),
        compiler_params=pltpu.CompilerParams(dimension_semantics=("arbitrary",)),
    )(gathered, bond_features, WbT, b_bond.reshape(1, _H),
      Wc, Wit, bgiA.reshape(1, 3 * _H), bhhn.reshape(1, _H))


def _readout_body(inv_n, asum_ref, msum_ref, wr1_ref, br1_ref, wr2_ref, br2_ref,
                  ws1_ref, bs1_ref, ws2_ref, bs2_ref, out_ref):
    tot = jnp.sum(asum_ref[...] + msum_ref[...], axis=0, keepdims=True)
    mol = tot * inv_n
    mol8 = jnp.broadcast_to(mol, (8, _H))
    h = jnp.dot(mol8, wr1_ref[...], preferred_element_type=jnp.float32)
    h = jnp.maximum(h + br1_ref[...], 0.0)
    m2 = jnp.dot(h, wr2_ref[...], preferred_element_type=jnp.float32) + br2_ref[...]
    h2 = jnp.dot(m2, ws1_ref[...], preferred_element_type=jnp.float32)
    h2 = jnp.maximum(h2 + bs1_ref[...], 0.0)
    sp = jnp.dot(h2, ws2_ref[...], preferred_element_type=jnp.float32) + bs2_ref[...]
    out_ref[...] = sp[0:1, :]


def _readout(asum, msum, n, W_r1, b_r1, W_r2, b_r2, W_s1, b_s1, W_s2, b_s2):
    spec_dim = W_s2.shape[0]
    h2 = W_s1.shape[0]
    return pl.pallas_call(
        functools.partial(_readout_body, 1.0 / n),
        in_specs=[pl.BlockSpec(a.shape, lambda: tuple(0 for _ in a.shape))
                  for a in (asum, msum, W_r1.T, b_r1.reshape(1, -1), W_r2.T,
                            b_r2.reshape(1, -1), W_s1.T, b_s1.reshape(1, -1),
                            W_s2.T, b_s2.reshape(1, -1))],
        out_specs=pl.BlockSpec((1, spec_dim), lambda: (0, 0)),
        out_shape=jax.ShapeDtypeStruct((1, spec_dim), jnp.float32),
    )(asum, msum, W_r1.T, b_r1.reshape(1, -1), W_r2.T, b_r2.reshape(1, -1),
      W_s1.T, b_s1.reshape(1, -1), W_s2.T, b_s2.reshape(1, -1))


def kernel(atom_features, bond_features, edge_src, edge_dst,
           W_atom, b_atom, W_bond, b_bond, W_msg, b_msg,
           gru_wih, gru_whh, gru_bih, gru_bhh,
           W_r1, b_r1, W_r2, b_r2, W_s1, b_s1, W_s2, b_s2):
    n = atom_features.shape[0]
    depth = 3

    Wc = W_msg.T @ gru_wih.T
    bc = b_msg @ gru_wih.T + gru_bih
    WhhT = gru_whh.T
    Wit = jnp.concatenate(
        [Wc[:, : 2 * _H] + WhhT[:, : 2 * _H], Wc[:, 2 * _H:],
         WhhT[:, 2 * _H:]], axis=1)
    bgiA = jnp.concatenate(
        [bc[: 2 * _H] + gru_bhh[: 2 * _H], bc[2 * _H:]])
    bhhn = gru_bhh[2 * _H:]

    av_packed, asum = _atom_encoder(atom_features, W_atom.T, b_atom)
    gathered = _gather_rows(av_packed, edge_src)
    msum = _mpnn(gathered, bond_features, W_bond.T, b_bond, Wc, Wit, bgiA,
                 bhhn, depth)
    return _readout(asum, msum, n, W_r1, b_r1, W_r2, b_r2,
                    W_s1, b_s1, W_s2, b_s2)
